# Initial kernel scaffold; baseline (speedup 1.0000x reference)
import jax
import jax.numpy as jnp
from jax import lax
from jax.experimental import pallas as pl
from jax.experimental.pallas import tpu as pltpu
from jax.experimental.pallas import tpu_sc as plsc

N_NODES = 10000
N_EDGES = 320000
D = 128
TW = 144
NC = 2
NS = 16
NW = NC * NS
EPT = N_EDGES // NW
CHUNK = 80
NCHUNK = EPT // CHUNK
RPT = N_NODES // NS
ZROWS = 125
BM = 1000


def _build_table(h, tab_ref):
    nrm = jnp.sqrt(jnp.sum(h * h, axis=1, keepdims=True)) + 1e-8
    hn = h / nrm
    tab_ref[:, 0:D] = hn
    tail = jnp.concatenate(
        [nrm, jnp.zeros((h.shape[0], TW - D - 1), jnp.float32)], axis=1)
    tab_ref[:, D:TW] = tail


def _proj_table_kernel(x_ref, w_ref, b_ref, tab_ref):
    h = jnp.dot(x_ref[...], w_ref[...],
                preferred_element_type=jnp.float32) + b_ref[...]
    _build_table(h, tab_ref)


def _combine(p0, p1):
    y = p0 + p1
    numer = y[:, 0:D]
    den = y[:, D:D + 1]
    return jnp.maximum(numer / (den + 1e-16), 0.0)


def _combine_proj_kernel(p0_ref, p1_ref, w_ref, b_ref, tab_ref):
    h1 = _combine(p0_ref[...], p1_ref[...])
    h = jnp.dot(h1, w_ref[...],
                preferred_element_type=jnp.float32) + b_ref[...]
    _build_table(h, tab_ref)


def _combine_out_kernel(p0_ref, p1_ref, out_ref):
    out_ref[...] = _combine(p0_ref[...], p1_ref[...])


def _proj_call(x, W, b):
    return pl.pallas_call(
        _proj_table_kernel,
        grid=(N_NODES // BM,),
        in_specs=[
            pl.BlockSpec((BM, D), lambda i: (i, 0)),
            pl.BlockSpec((D, D), lambda i: (0, 0)),
            pl.BlockSpec((1, D), lambda i: (0, 0)),
        ],
        out_specs=pl.BlockSpec((BM, TW), lambda i: (i, 0)),
        out_shape=jax.ShapeDtypeStruct((N_NODES, TW), jnp.float32),
    )(x, W, b.reshape(1, D))


def _combine_proj_call(p0, p1, W, b):
    return pl.pallas_call(
        _combine_proj_kernel,
        grid=(N_NODES // BM,),
        in_specs=[
            pl.BlockSpec((BM, TW), lambda i: (i, 0)),
            pl.BlockSpec((BM, TW), lambda i: (i, 0)),
            pl.BlockSpec((D, D), lambda i: (0, 0)),
            pl.BlockSpec((1, D), lambda i: (0, 0)),
        ],
        out_specs=pl.BlockSpec((BM, TW), lambda i: (i, 0)),
        out_shape=jax.ShapeDtypeStruct((N_NODES, TW), jnp.float32),
    )(p0, p1, W, b.reshape(1, D))


def _combine_out_call(p0, p1):
    return pl.pallas_call(
        _combine_out_kernel,
        grid=(N_NODES // BM,),
        in_specs=[
            pl.BlockSpec((BM, TW), lambda i: (i, 0)),
            pl.BlockSpec((BM, TW), lambda i: (i, 0)),
        ],
        out_specs=pl.BlockSpec((BM, D), lambda i: (i, 0)),
        out_shape=jax.ShapeDtypeStruct((N_NODES, D), jnp.float32),
    )(p0, p1)


def _edge_kernel_body(tab, srce, dste, betav, out,
                      srcbuf, dstbuf, sidx, didx, betabuf,
                      dotbuf, npbuf, sbuf, wbuf, zbuf, accum, sem_s, sem_d):
    c = lax.axis_index("c")
    s = lax.axis_index("s")
    tile_base = (c * NS + s) * EPT

    pltpu.sync_copy(betav, betabuf)
    beta = betabuf[...]

    def _zero_body(i, carry):
        zbuf[i // (TW // 16), pl.ds((i % (TW // 16)) * 16, 16)] = (
            jnp.zeros((16,), jnp.float32))
        return carry
    lax.fori_loop(0, ZROWS * (TW // 16), _zero_body, 0)
    for k in range(RPT // ZROWS):
        pltpu.sync_copy(zbuf, accum.at[pl.ds(s * RPT + k * ZROWS, ZROWS)])
    plsc.subcore_barrier()

    def chunk_body(ci, carry):
        base = tile_base + ci * CHUNK
        pltpu.sync_copy(srce.at[pl.ds(base, CHUNK)], sidx)
        pltpu.sync_copy(dste.at[pl.ds(base, CHUNK)], didx)
        cp_s = pltpu.async_copy(tab.at[sidx], srcbuf, sem_s)
        cp_d = pltpu.async_copy(tab.at[didx], dstbuf, sem_d)
        cp_s.wait()
        cp_d.wait()

        def dot_body(e, cc):
            acc = srcbuf[e, pl.ds(0, 16)] * dstbuf[e, pl.ds(0, 16)]
            for k in range(1, D // 16):
                acc = acc + (srcbuf[e, pl.ds(k * 16, 16)] *
                             dstbuf[e, pl.ds(k * 16, 16)])
            dotbuf[e] = jnp.sum(acc)
            npbuf[e] = srcbuf[e, D]
            return cc
        lax.fori_loop(0, CHUNK, dot_body, 0)

        for g in range(CHUNK // 16):
            dv = dotbuf[pl.ds(g * 16, 16)]
            npv = npbuf[pl.ds(g * 16, 16)]
            wv = jnp.exp(beta * dv)
            sbuf[pl.ds(g * 16, 16)] = wv * npv
            wbuf[pl.ds(g * 16, 16)] = wv

        def scale_body(e, cc):
            se = sbuf[e]
            for k in range(D // 16):
                v = srcbuf[e, pl.ds(k * 16, 16)]
                srcbuf[e, pl.ds(k * 16, 16)] = v * se
            srcbuf[e, D] = wbuf[e]
            return cc
        lax.fori_loop(0, CHUNK, scale_body, 0)

        pltpu.sync_copy(srcbuf, accum.at[didx], add=True)
        return carry
    lax.fori_loop(0, NCHUNK, chunk_body, 0)

    plsc.subcore_barrier()
    for k in range(RPT // ZROWS):
        rows = pl.ds(s * RPT + k * ZROWS, ZROWS)
        pltpu.sync_copy(accum.at[rows], zbuf)
        pltpu.sync_copy(zbuf, out.at[c, rows])


def _make_edge_call():
    mesh = plsc.VectorSubcoreMesh(
        core_axis_name="c", subcore_axis_name="s",
        num_cores=NC, num_subcores=NS)
    return pl.kernel(
        _edge_kernel_body,
        out_type=jax.ShapeDtypeStruct((NC, N_NODES, TW), jnp.float32),
        mesh=mesh,
        scratch_types=[
            pltpu.VMEM((CHUNK, TW), jnp.float32),
            pltpu.VMEM((CHUNK, TW), jnp.float32),
            pltpu.VMEM((CHUNK,), jnp.int32),
            pltpu.VMEM((CHUNK,), jnp.int32),
            pltpu.VMEM((16,), jnp.float32),
            pltpu.VMEM((CHUNK,), jnp.float32),
            pltpu.VMEM((CHUNK,), jnp.float32),
            pltpu.VMEM((CHUNK,), jnp.float32),
            pltpu.VMEM((CHUNK,), jnp.float32),
            pltpu.VMEM((ZROWS, TW), jnp.float32),
            pltpu.VMEM_SHARED((N_NODES, TW), jnp.float32),
            pltpu.SemaphoreType.DMA,
            pltpu.SemaphoreType.DMA,
        ],
    )


def kernel(x, edge_index, W1, b1, beta1, W2, b2, beta2):
    ei = edge_index.astype(jnp.int32)
    srce = ei[0]
    dste = ei[1]
    edge_call = _make_edge_call()

    tab1 = _proj_call(x, W1, b1)
    p1 = edge_call(tab1, srce, dste, jnp.full((16,), beta1, jnp.float32))
    tab2 = _combine_proj_call(p1[0], p1[1], W2, b2)
    p2 = edge_call(tab2, srce, dste, jnp.full((16,), beta2, jnp.float32))
    return _combine_out_call(p2[0], p2[1])


# speedup vs baseline: 11.2957x; 11.2957x over previous
import jax
import jax.numpy as jnp
from jax import lax
from jax.experimental import pallas as pl
from jax.experimental.pallas import tpu as pltpu
from jax.experimental.pallas import tpu_sc as plsc

N_NODES = 10000
N_EDGES = 320000
D = 128
NPW = 16
NC = 2
NS = 16
NW = NC * NS
EPT = N_EDGES // NW
CHUNK = 80
NCHUNK = EPT // CHUNK
RPT = N_NODES // NS
ZROWS = 125
DEN_ROWS = N_NODES // 2
DRPT = 312
BM = 1000


def _build_tables(h, hn_ref, np_ref):
    nrm = jnp.sqrt(jnp.sum(h * h, axis=1, keepdims=True)) + 1e-8
    hn_ref[...] = h / nrm
    lanes = lax.broadcasted_iota(jnp.int32, (h.shape[0], NPW), 1)
    np_ref[...] = jnp.where(lanes == 0, nrm, 0.0)


def _proj_table_kernel(x_ref, w_ref, b_ref, hn_ref, np_ref):
    h = jnp.dot(x_ref[...], w_ref[...],
                preferred_element_type=jnp.float32) + b_ref[...]
    _build_tables(h, hn_ref, np_ref)


def _combine(pn0, pn1, pd0, pd1):
    numer = pn0 + pn1
    den = (pd0 + pd1)[:, 0:1]
    return jnp.maximum(numer / (den + 1e-16), 0.0)


def _combine_proj_kernel(pn0_ref, pn1_ref, pd0_ref, pd1_ref, w_ref, b_ref,
                         hn_ref, np_ref):
    h1 = _combine(pn0_ref[...], pn1_ref[...], pd0_ref[...], pd1_ref[...])
    h = jnp.dot(h1, w_ref[...],
                preferred_element_type=jnp.float32) + b_ref[...]
    _build_tables(h, hn_ref, np_ref)


def _combine_out_kernel(pn0_ref, pn1_ref, pd0_ref, pd1_ref, out_ref):
    out_ref[...] = _combine(pn0_ref[...], pn1_ref[...],
                            pd0_ref[...], pd1_ref[...])


_TAB_OUT = (
    jax.ShapeDtypeStruct((N_NODES, D), jnp.float32),
    jax.ShapeDtypeStruct((N_NODES, NPW), jnp.float32),
)
_TAB_OUT_SPECS = (
    pl.BlockSpec((BM, D), lambda i: (i, 0)),
    pl.BlockSpec((BM, NPW), lambda i: (i, 0)),
)


def _proj_call(x, W, b):
    return pl.pallas_call(
        _proj_table_kernel,
        grid=(N_NODES // BM,),
        in_specs=[
            pl.BlockSpec((BM, D), lambda i: (i, 0)),
            pl.BlockSpec((D, D), lambda i: (0, 0)),
            pl.BlockSpec((1, D), lambda i: (0, 0)),
        ],
        out_specs=_TAB_OUT_SPECS,
        out_shape=_TAB_OUT,
    )(x, W, b.reshape(1, D))


def _combine_proj_call(pn0, pn1, pd0, pd1, W, b):
    return pl.pallas_call(
        _combine_proj_kernel,
        grid=(N_NODES // BM,),
        in_specs=[
            pl.BlockSpec((BM, D), lambda i: (i, 0)),
            pl.BlockSpec((BM, D), lambda i: (i, 0)),
            pl.BlockSpec((BM, 8), lambda i: (i, 0)),
            pl.BlockSpec((BM, 8), lambda i: (i, 0)),
            pl.BlockSpec((D, D), lambda i: (0, 0)),
            pl.BlockSpec((1, D), lambda i: (0, 0)),
        ],
        out_specs=_TAB_OUT_SPECS,
        out_shape=_TAB_OUT,
    )(pn0, pn1, pd0, pd1, W, b.reshape(1, D))


def _combine_out_call(pn0, pn1, pd0, pd1):
    return pl.pallas_call(
        _combine_out_kernel,
        grid=(N_NODES // BM,),
        in_specs=[
            pl.BlockSpec((BM, D), lambda i: (i, 0)),
            pl.BlockSpec((BM, D), lambda i: (i, 0)),
            pl.BlockSpec((BM, 8), lambda i: (i, 0)),
            pl.BlockSpec((BM, 8), lambda i: (i, 0)),
        ],
        out_specs=pl.BlockSpec((BM, D), lambda i: (i, 0)),
        out_shape=jax.ShapeDtypeStruct((N_NODES, D), jnp.float32),
    )(pn0, pn1, pd0, pd1)


def _edge_kernel_body(hn_tab, np_tab, srce, dste, betav, out_n, out_d,
                      srcbuf, dstbuf, npgbuf, denbuf, sidx, didx, didx2,
                      parbuf, betabuf, sbuf, wbuf, zbuf, dzbuf,
                      numer, den, sem_s, sem_d, sem_n):
    c = lax.axis_index("c")
    s = lax.axis_index("s")
    tile_base = (c * NS + s) * EPT
    lane = lax.iota(jnp.int32, 16)

    pltpu.sync_copy(betav, betabuf)
    beta = betabuf[...]

    def _zero_body(i, carry):
        zbuf[i // (D // 16), pl.ds((i % (D // 16)) * 16, 16)] = (
            jnp.zeros((16,), jnp.float32))
        return carry
    lax.fori_loop(0, ZROWS * (D // 16), _zero_body, 0)

    def _dzero_body(i, carry):
        dzbuf[i, pl.ds(0, 16)] = jnp.zeros((16,), jnp.float32)
        return carry
    lax.fori_loop(0, DRPT, _dzero_body, 0)

    for k in range(RPT // ZROWS):
        pltpu.sync_copy(zbuf, numer.at[pl.ds(s * RPT + k * ZROWS, ZROWS)])
    pltpu.sync_copy(dzbuf, den.at[pl.ds(s * DRPT, DRPT)])

    @pl.when(s == 0)
    def _zero_den_tail():
        pltpu.sync_copy(dzbuf.at[pl.ds(0, DEN_ROWS - NS * DRPT)],
                        den.at[pl.ds(NS * DRPT, DEN_ROWS - NS * DRPT)])

    plsc.subcore_barrier()

    def chunk_body(ci, carry):
        base = tile_base + ci * CHUNK
        pltpu.sync_copy(srce.at[pl.ds(base, CHUNK)], sidx)
        pltpu.sync_copy(dste.at[pl.ds(base, CHUNK)], didx)
        cp_s = pltpu.async_copy(hn_tab.at[sidx], srcbuf, sem_s)
        cp_d = pltpu.async_copy(hn_tab.at[didx], dstbuf, sem_d)
        cp_n = pltpu.async_copy(np_tab.at[sidx], npgbuf, sem_n)
        cp_s.wait()
        cp_d.wait()
        cp_n.wait()

        for g in range(CHUNK // 16):
            def dot16(e16, dv, g=g):
                e = g * 16 + e16
                acc = srcbuf[e, pl.ds(0, 16)] * dstbuf[e, pl.ds(0, 16)]
                for k in range(1, D // 16):
                    acc = acc + (srcbuf[e, pl.ds(k * 16, 16)] *
                                 dstbuf[e, pl.ds(k * 16, 16)])
                return jnp.where(lane == e16, jnp.sum(acc), dv)
            dv = lax.fori_loop(0, 16, dot16, jnp.zeros((16,), jnp.float32))
            rows = lane + g * 16
            npv = plsc.load_gather(
                npgbuf, [rows, jnp.zeros((16,), jnp.int32)])
            wv = jnp.exp(beta * dv)
            sbuf[pl.ds(g * 16, 16)] = wv * npv
            wbuf[pl.ds(g * 16, 16)] = wv
            dvi = didx[pl.ds(g * 16, 16)]
            didx2[pl.ds(g * 16, 16)] = lax.shift_right_logical(dvi, 1)
            parbuf[pl.ds(g * 16, 16)] = (dvi & 1) * 8

        def scale_body(e, cc):
            se = sbuf[pl.ds(e, 16)][0]
            we = wbuf[pl.ds(e, 16)][0]
            pe = parbuf[pl.ds(e, 16)][0]
            for k in range(D // 16):
                v = srcbuf[e, pl.ds(k * 16, 16)]
                srcbuf[e, pl.ds(k * 16, 16)] = v * se
            denbuf[e, pl.ds(0, 16)] = jnp.where(
                lane == pe, we, jnp.zeros((16,), jnp.float32))
            return cc
        lax.fori_loop(0, CHUNK, scale_body, 0)

        pltpu.sync_copy(srcbuf, numer.at[didx], add=True)
        pltpu.sync_copy(denbuf, den.at[didx2], add=True)
        return carry
    lax.fori_loop(0, NCHUNK, chunk_body, 0)

    plsc.subcore_barrier()
    for k in range(RPT // ZROWS):
        rows = pl.ds(s * RPT + k * ZROWS, ZROWS)
        pltpu.sync_copy(numer.at[rows], zbuf)
        pltpu.sync_copy(zbuf, out_n.at[c, rows])
    drows = pl.ds(s * DRPT, DRPT)
    pltpu.sync_copy(den.at[drows], dzbuf)
    pltpu.sync_copy(dzbuf, out_d.at[c, drows])

    @pl.when(s == 0)
    def _export_den_tail():
        tail = pl.ds(NS * DRPT, DEN_ROWS - NS * DRPT)
        pltpu.sync_copy(den.at[tail], dzbuf.at[pl.ds(0, DEN_ROWS - NS * DRPT)])
        pltpu.sync_copy(dzbuf.at[pl.ds(0, DEN_ROWS - NS * DRPT)],
                        out_d.at[c, tail])


def _make_edge_call():
    mesh = plsc.VectorSubcoreMesh(
        core_axis_name="c", subcore_axis_name="s",
        num_cores=NC, num_subcores=NS)
    return pl.kernel(
        _edge_kernel_body,
        out_type=(
            jax.ShapeDtypeStruct((NC, N_NODES, D), jnp.float32),
            jax.ShapeDtypeStruct((NC, DEN_ROWS, NPW), jnp.float32),
        ),
        mesh=mesh,
        compiler_params=pltpu.CompilerParams(
            use_tc_tiling_on_sc=False, needs_layout_passes=False),
        scratch_types=[
            pltpu.VMEM((CHUNK, D), jnp.float32),
            pltpu.VMEM((CHUNK, D), jnp.float32),
            pltpu.VMEM((CHUNK, NPW), jnp.float32),
            pltpu.VMEM((CHUNK, NPW), jnp.float32),
            pltpu.VMEM((CHUNK,), jnp.int32),
            pltpu.VMEM((CHUNK,), jnp.int32),
            pltpu.VMEM((CHUNK,), jnp.int32),
            pltpu.VMEM((CHUNK + 16,), jnp.int32),
            pltpu.VMEM((16,), jnp.float32),
            pltpu.VMEM((CHUNK + 16,), jnp.float32),
            pltpu.VMEM((CHUNK + 16,), jnp.float32),
            pltpu.VMEM((ZROWS, D), jnp.float32),
            pltpu.VMEM((DRPT, NPW), jnp.float32),
            pltpu.VMEM_SHARED((N_NODES, D), jnp.float32),
            pltpu.VMEM_SHARED((DEN_ROWS, NPW), jnp.float32),
            pltpu.SemaphoreType.DMA,
            pltpu.SemaphoreType.DMA,
            pltpu.SemaphoreType.DMA,
        ],
    )


def kernel(x, edge_index, W1, b1, beta1, W2, b2, beta2):
    ei = edge_index.astype(jnp.int32)
    srce = ei[0]
    dste = ei[1]
    edge_call = _make_edge_call()

    hn1, np1 = _proj_call(x, W1, b1)
    pn1, pd1 = edge_call(hn1, np1, srce, dste,
                         jnp.full((16,), beta1, jnp.float32))
    pd1 = pd1.reshape(NC, N_NODES, 8)
    hn2, np2 = _combine_proj_call(pn1[0], pn1[1], pd1[0], pd1[1], W2, b2)
    pn2, pd2 = edge_call(hn2, np2, srce, dste,
                         jnp.full((16,), beta2, jnp.float32))
    pd2 = pd2.reshape(NC, N_NODES, 8)
    return _combine_out_call(pn2[0], pn2[1], pd2[0], pd2[1])
